# Initial kernel scaffold; baseline (speedup 1.0000x reference)
import jax
import jax.numpy as jnp
from jax.experimental import pallas as pl


def kernel(x, Wg, bg, W1, b1, W2, b2):
    raise NotImplementedError("write your pallas kernel here")


# speedup vs baseline: 1.7638x; 1.7638x over previous
import functools

import jax
import jax.numpy as jnp
from jax import lax
from jax.experimental import pallas as pl
from jax.experimental.pallas import tpu as pltpu
from jax.experimental.pallas import tpu_sc as plsc

D = 1024
E = 8
DFF = 4096
N_TOK = 8192
BT = 1024
NB = N_TOK // BT
A = 2 * N_TOK
TB = 512
NT = A // TB
NW = NT + E - 1
F = 1024
NF = DFF // F

_NEG = -3.0e38


def _gating_body(x_ref, wg_ref, bg_ref, i0_ref, i1_ref, r0_ref, r1_ref,
                 w0_ref, w1_ref, hist_ref, cnt_ref):
    b = pl.program_id(0)

    @pl.when(b == 0)
    def _():
        cnt_ref[...] = jnp.zeros((1, E), jnp.float32)

    x = x_ref[...]
    logits = jax.lax.dot_general(
        x, wg_ref[...], (((1,), (1,)), ((), ())),
        preferred_element_type=jnp.float32) + bg_ref[...]

    eidx = lax.broadcasted_iota(jnp.int32, (BT, E), 1)
    m1 = jnp.max(logits, axis=1, keepdims=True)
    i1 = jnp.min(jnp.where(logits == m1, eidx, E), axis=1)
    l2 = jnp.where(eidx == i1[:, None], _NEG, logits)
    m2 = jnp.max(l2, axis=1, keepdims=True)
    i2 = jnp.min(jnp.where(l2 == m2, eidx, E), axis=1)

    w0 = 1.0 / (1.0 + jnp.exp(m2[:, 0] - m1[:, 0]))
    w1 = 1.0 - w0

    o0 = (eidx == i1[:, None]).astype(jnp.float32)
    o1 = (eidx == i2[:, None]).astype(jnp.float32)
    osum = o0 + o1
    ti = lax.broadcasted_iota(jnp.int32, (BT, BT), 0)
    tj = lax.broadcasted_iota(jnp.int32, (BT, BT), 1)
    tri = (tj < ti).astype(jnp.float32)
    s = jax.lax.dot_general(tri, osum, (((1,), (0,)), ((), ())),
                            preferred_element_type=jnp.float32)
    cnt = cnt_ref[...]
    r0 = jnp.sum(o0 * (s + cnt), axis=1)
    r1 = jnp.sum(o1 * (s + o0 + cnt), axis=1)
    cnt = cnt + jnp.sum(osum, axis=0, keepdims=True)
    cnt_ref[...] = cnt

    i0_ref[...] = i1.reshape(1, 1, BT)
    i1_ref[...] = i2.reshape(1, 1, BT)
    r0_ref[...] = r0.astype(jnp.int32).reshape(1, 1, BT)
    r1_ref[...] = r1.astype(jnp.int32).reshape(1, 1, BT)
    w0_ref[...] = w0.reshape(1, 1, BT)
    w1_ref[...] = w1.reshape(1, 1, BT)
    hist_ref[...] = jnp.concatenate(
        [cnt.astype(jnp.int32), jnp.zeros((1, 16 - E), jnp.int32)], axis=1)


def _gating_call(xf, Wg, bg):
    outs = [
        jax.ShapeDtypeStruct((NB, 1, BT), jnp.int32),
        jax.ShapeDtypeStruct((NB, 1, BT), jnp.int32),
        jax.ShapeDtypeStruct((NB, 1, BT), jnp.int32),
        jax.ShapeDtypeStruct((NB, 1, BT), jnp.int32),
        jax.ShapeDtypeStruct((NB, 1, BT), jnp.float32),
        jax.ShapeDtypeStruct((NB, 1, BT), jnp.float32),
        jax.ShapeDtypeStruct((1, 16), jnp.int32),
    ]
    blk = [pl.BlockSpec((1, 1, BT), lambda b: (b, 0, 0))] * 6 + [
        pl.BlockSpec((1, 16), lambda b: (0, 0))]
    return pl.pallas_call(
        _gating_body,
        grid=(NB,),
        in_specs=[
            pl.BlockSpec((BT, D), lambda b: (b, 0)),
            pl.BlockSpec((E, D), lambda b: (0, 0)),
            pl.BlockSpec((E,), lambda b: (0,)),
        ],
        out_specs=blk,
        out_shape=outs,
        scratch_shapes=[pltpu.VMEM((1, E), jnp.float32)],
    )(xf, Wg, bg)


def _sc_mesh():
    return plsc.VectorSubcoreMesh(core_axis_name="c", subcore_axis_name="s")
_NTILES = 32
_CH = 32
_TPT = N_TOK // _NTILES


def _dispatch_body(x_hbm, i0_hbm, i1_hbm, r0_hbm, r1_hbm, offs_hbm,
                   xs_hbm, p0_hbm, p1_hbm,
                   obuf, ibuf, rbuf, p0buf, p1buf, xbuf, sem):
    wid = lax.axis_index("s") * 2 + lax.axis_index("c")
    tok0 = wid * _TPT

    pltpu.sync_copy(offs_hbm, obuf)

    def chunk(c, carry):
        base = tok0 + c * _CH
        for ibh, rbh, pbuf in ((i0_hbm, r0_hbm, p0buf), (i1_hbm, r1_hbm, p1buf)):
            pltpu.sync_copy(ibh.at[pl.ds(base, _CH)], ibuf)
            pltpu.sync_copy(rbh.at[pl.ds(base, _CH)], rbuf)
            for j in range(_CH // 16):
                e16 = ibuf[pl.ds(j * 16, 16)]
                r16 = rbuf[pl.ds(j * 16, 16)]
                o16 = plsc.load_gather(obuf, [e16])
                pbuf[pl.ds(j * 16, 16)] = r16 + o16
        pltpu.sync_copy(p0buf, p0_hbm.at[pl.ds(base, _CH)])
        pltpu.sync_copy(p1buf, p1_hbm.at[pl.ds(base, _CH)])
        pltpu.sync_copy(x_hbm.at[pl.ds(base, _CH)], xbuf)
        pltpu.async_copy(xbuf, xs_hbm.at[p0buf], sem).wait()
        pltpu.async_copy(xbuf, xs_hbm.at[p1buf], sem).wait()
        return carry

    lax.fori_loop(0, _TPT // _CH, chunk, 0)


def _dispatch_call(xf, i0, i1, r0, r1, offs):
    return pl.kernel(
        _dispatch_body,
        out_type=[
            jax.ShapeDtypeStruct((A, D), jnp.float32),
            jax.ShapeDtypeStruct((N_TOK,), jnp.int32),
            jax.ShapeDtypeStruct((N_TOK,), jnp.int32),
        ],
        mesh=_sc_mesh(),
        compiler_params=pltpu.CompilerParams(needs_layout_passes=False),
        scratch_types=[
            pltpu.VMEM((16,), jnp.int32),
            pltpu.VMEM((_CH,), jnp.int32),
            pltpu.VMEM((_CH,), jnp.int32),
            pltpu.VMEM((_CH,), jnp.int32),
            pltpu.VMEM((_CH,), jnp.int32),
            pltpu.VMEM((_CH, D), jnp.float32),
            pltpu.SemaphoreType.DMA,
        ],
    )(xf, i0, i1, r0, r1, offs)


def _ffn_body(t_ref, e_ref, lo_ref, hi_ref,
              xs_ref, w1_ref, b1_ref, w2_ref, b2_ref, out_ref, acc_ref):
    k = pl.program_id(0)
    f = pl.program_id(1)
    lo = lo_ref[k]
    hi = hi_ref[k]

    @pl.when(hi > lo)
    def _():
        xb = xs_ref[...].astype(jnp.bfloat16)
        w1b = w1_ref[0]
        h = jax.lax.dot_general(xb, w1b, (((1,), (1,)), ((), ())),
                                preferred_element_type=jnp.float32)
        h = jnp.maximum(h + b1_ref[0, 0], 0.0).astype(jnp.bfloat16)
        w2b = w2_ref[0]
        y = jax.lax.dot_general(h, w2b, (((1,), (1,)), ((), ())),
                                preferred_element_type=jnp.float32)

        @pl.when(f == 0)
        def _():
            acc_ref[...] = y

        @pl.when(f > 0)
        def _():
            acc_ref[...] = acc_ref[...] + y

        @pl.when(f == NF - 1)
        def _():
            t = t_ref[k]
            row = t * TB + lax.broadcasted_iota(jnp.int32, (TB, 1), 0)
            valid = (row >= lo) & (row < hi)
            yout = acc_ref[...] + b2_ref[0]
            out_ref[...] = jnp.where(valid, yout, out_ref[...])


def _ffn_call(wt, we, wlo, whi, xs, W1b, b1, W2b, b2):
    grid_spec = pltpu.PrefetchScalarGridSpec(
        num_scalar_prefetch=4,
        grid=(NW, NF),
        in_specs=[
            pl.BlockSpec((TB, D), lambda k, f, t, e, lo, hi: (t[k], 0)),
            pl.BlockSpec((1, F, D), lambda k, f, t, e, lo, hi: (e[k], f, 0)),
            pl.BlockSpec((1, 1, 1, F), lambda k, f, t, e, lo, hi: (e[k], f, 0, 0)),
            pl.BlockSpec((1, D, F), lambda k, f, t, e, lo, hi: (e[k], 0, f)),
            pl.BlockSpec((1, 1, D), lambda k, f, t, e, lo, hi: (e[k], 0, 0)),
        ],
        out_specs=pl.BlockSpec((TB, D), lambda k, f, t, e, lo, hi: (t[k], 0)),
        scratch_shapes=[pltpu.VMEM((TB, D), jnp.float32)],
    )
    return pl.pallas_call(
        _ffn_body,
        grid_spec=grid_spec,
        out_shape=jax.ShapeDtypeStruct((A, D), jnp.float32),
    )(wt, we, wlo, whi, xs, W1b, b1, W2b, b2)


_CC = 16


def _combine_body(ys_hbm, p0_hbm, p1_hbm, w0_hbm, w1_hbm, out_hbm,
                  pbuf0, pbuf1, abuf, bbuf, obuf, wb0, wb1, sem):
    wid = lax.axis_index("s") * 2 + lax.axis_index("c")
    tok0 = wid * _TPT

    def chunk(c, carry):
        base = tok0 + c * _CC
        pltpu.sync_copy(p0_hbm.at[pl.ds(base, _CC)], pbuf0)
        pltpu.sync_copy(p1_hbm.at[pl.ds(base, _CC)], pbuf1)
        pltpu.sync_copy(w0_hbm.at[pl.ds(base, _CC)], wb0)
        pltpu.sync_copy(w1_hbm.at[pl.ds(base, _CC)], wb1)
        pltpu.async_copy(ys_hbm.at[pbuf0], abuf, sem).wait()
        pltpu.async_copy(ys_hbm.at[pbuf1], bbuf, sem).wait()

        def row(r, carry2):
            ridx = jnp.broadcast_to(r, (16,)).astype(jnp.int32)
            w0v = plsc.load_gather(wb0, [ridx])
            w1v = plsc.load_gather(wb1, [ridx])

            def vec(j, carry3):
                av = abuf[r, pl.ds(j * 16, 16)]
                bv = bbuf[r, pl.ds(j * 16, 16)]
                obuf[r, pl.ds(j * 16, 16)] = av * w0v + bv * w1v
                return carry3

            return lax.fori_loop(0, D // 16, vec, carry2, unroll=4)

        lax.fori_loop(0, _CC, row, 0)
        pltpu.sync_copy(obuf, out_hbm.at[pl.ds(base, _CC)])
        return carry

    lax.fori_loop(0, _TPT // _CC, chunk, 0)


def _combine_call(ys, p0, p1, w0, w1):
    return pl.kernel(
        _combine_body,
        out_type=jax.ShapeDtypeStruct((N_TOK, D), jnp.float32),
        mesh=_sc_mesh(),
        compiler_params=pltpu.CompilerParams(needs_layout_passes=False),
        scratch_types=[
            pltpu.VMEM((_CC,), jnp.int32),
            pltpu.VMEM((_CC,), jnp.int32),
            pltpu.VMEM((_CC, D), jnp.float32),
            pltpu.VMEM((_CC, D), jnp.float32),
            pltpu.VMEM((_CC, D), jnp.float32),
            pltpu.VMEM((_CC,), jnp.float32),
            pltpu.VMEM((_CC,), jnp.float32),
            pltpu.SemaphoreType.DMA,
        ],
    )(ys, p0, p1, w0, w1)


def _schedule(hist):
    off = jnp.concatenate(
        [jnp.zeros((1,), jnp.int32), jnp.cumsum(hist[0, :E], dtype=jnp.int32)])
    c = jnp.arange(NT * E, dtype=jnp.int32)
    t = c // E
    e = c % E
    lo = jnp.maximum(t * TB, off[e])
    hi = jnp.minimum((t + 1) * TB, off[e + 1])
    valid = hi > lo
    slot = jnp.where(valid, jnp.cumsum(valid.astype(jnp.int32)) - 1, NW)
    nvalid = jnp.sum(valid.astype(jnp.int32))
    wt = jnp.zeros((NW + 1,), jnp.int32).at[slot].set(t, mode="drop")
    we = jnp.zeros((NW + 1,), jnp.int32).at[slot].set(e, mode="drop")
    wlo = jnp.zeros((NW + 1,), jnp.int32).at[slot].set(lo, mode="drop")
    whi = jnp.zeros((NW + 1,), jnp.int32).at[slot].set(hi, mode="drop")
    pad = jnp.arange(NW) >= nvalid
    lt = wt[jnp.maximum(nvalid - 1, 0)]
    le = we[jnp.maximum(nvalid - 1, 0)]
    wt = jnp.where(pad, lt, wt[:NW])
    we = jnp.where(pad, le, we[:NW])
    wlo = jnp.where(pad, 0, wlo[:NW])
    whi = jnp.where(pad, 0, whi[:NW])
    offs16 = jnp.concatenate([off[:E], jnp.zeros((16 - E,), jnp.int32)])
    return wt, we, wlo, whi, offs16


def kernel(x, Wg, bg, W1, b1, W2, b2):
    xf = x.reshape(N_TOK, D)
    i0, i1, r0, r1, w0, w1, hist = _gating_call(xf, Wg, bg)
    wt, we, wlo, whi, offs16 = _schedule(hist)
    xs, p0, p1 = _dispatch_call(
        xf, i0.reshape(-1), i1.reshape(-1), r0.reshape(-1), r1.reshape(-1),
        offs16)
    ys = _ffn_call(wt, we, wlo, whi, xs,
                   W1.astype(jnp.bfloat16), b1.reshape(E, NF, 1, F),
                   W2.astype(jnp.bfloat16), b2.reshape(E, 1, D))
    out = _combine_call(ys, p0, p1, w0.reshape(-1), w1.reshape(-1))
    return out.reshape(x.shape)


# speedup vs baseline: 1.9063x; 1.0808x over previous
import functools

import jax
import jax.numpy as jnp
from jax import lax
from jax.experimental import pallas as pl
from jax.experimental.pallas import tpu as pltpu
from jax.experimental.pallas import tpu_sc as plsc

D = 1024
E = 8
DFF = 4096
N_TOK = 8192
BT = 1024
NB = N_TOK // BT
A = 2 * N_TOK
TB = 512
NT = A // TB
NW = NT + E - 1
F = 1024
NF = DFF // F

_NEG = -3.0e38


def _gating_body(x_ref, wg_ref, bg_ref, i0_ref, i1_ref, r0_ref, r1_ref,
                 w0_ref, w1_ref, hist_ref, cnt_ref, tri_ref):
    b = pl.program_id(0)

    @pl.when(b == 0)
    def _():
        cnt_ref[...] = jnp.zeros((1, E), jnp.float32)
        ti = lax.broadcasted_iota(jnp.int32, (BT, BT), 0)
        tj = lax.broadcasted_iota(jnp.int32, (BT, BT), 1)
        tri_ref[...] = (tj < ti).astype(jnp.float32)

    x = x_ref[...]
    logits = jax.lax.dot_general(
        x, wg_ref[...], (((1,), (1,)), ((), ())),
        preferred_element_type=jnp.float32) + bg_ref[...]

    eidx = lax.broadcasted_iota(jnp.int32, (BT, E), 1)
    m1 = jnp.max(logits, axis=1, keepdims=True)
    i1 = jnp.min(jnp.where(logits == m1, eidx, E), axis=1)
    l2 = jnp.where(eidx == i1[:, None], _NEG, logits)
    m2 = jnp.max(l2, axis=1, keepdims=True)
    i2 = jnp.min(jnp.where(l2 == m2, eidx, E), axis=1)

    w0 = 1.0 / (1.0 + jnp.exp(m2[:, 0] - m1[:, 0]))
    w1 = 1.0 - w0

    o0 = (eidx == i1[:, None]).astype(jnp.float32)
    o1 = (eidx == i2[:, None]).astype(jnp.float32)
    osum = o0 + o1
    s = jax.lax.dot_general(tri_ref[...], osum, (((1,), (0,)), ((), ())),
                            preferred_element_type=jnp.float32)
    cnt = cnt_ref[...]
    r0 = jnp.sum(o0 * (s + cnt), axis=1)
    r1 = jnp.sum(o1 * (s + o0 + cnt), axis=1)
    cnt = cnt + jnp.sum(osum, axis=0, keepdims=True)
    cnt_ref[...] = cnt

    i0_ref[...] = i1.reshape(1, 1, BT)
    i1_ref[...] = i2.reshape(1, 1, BT)
    r0_ref[...] = r0.astype(jnp.int32).reshape(1, 1, BT)
    r1_ref[...] = r1.astype(jnp.int32).reshape(1, 1, BT)
    w0_ref[...] = w0.reshape(1, 1, BT)
    w1_ref[...] = w1.reshape(1, 1, BT)
    hist_ref[...] = jnp.concatenate(
        [cnt.astype(jnp.int32), jnp.zeros((1, 16 - E), jnp.int32)], axis=1)


def _gating_call(xf, Wg, bg):
    outs = [
        jax.ShapeDtypeStruct((NB, 1, BT), jnp.int32),
        jax.ShapeDtypeStruct((NB, 1, BT), jnp.int32),
        jax.ShapeDtypeStruct((NB, 1, BT), jnp.int32),
        jax.ShapeDtypeStruct((NB, 1, BT), jnp.int32),
        jax.ShapeDtypeStruct((NB, 1, BT), jnp.float32),
        jax.ShapeDtypeStruct((NB, 1, BT), jnp.float32),
        jax.ShapeDtypeStruct((1, 16), jnp.int32),
    ]
    blk = [pl.BlockSpec((1, 1, BT), lambda b: (b, 0, 0))] * 6 + [
        pl.BlockSpec((1, 16), lambda b: (0, 0))]
    return pl.pallas_call(
        _gating_body,
        grid=(NB,),
        in_specs=[
            pl.BlockSpec((BT, D), lambda b: (b, 0)),
            pl.BlockSpec((E, D), lambda b: (0, 0)),
            pl.BlockSpec((E,), lambda b: (0,)),
        ],
        out_specs=blk,
        out_shape=outs,
        scratch_shapes=[pltpu.VMEM((1, E), jnp.float32),
                        pltpu.VMEM((BT, BT), jnp.float32)],
    )(xf, Wg, bg)


def _sc_mesh():
    return plsc.VectorSubcoreMesh(core_axis_name="c", subcore_axis_name="s")
_NTILES = 32
_CH = 32
_TPT = N_TOK // _NTILES


def _dispatch_body(x_hbm, i0_hbm, i1_hbm, r0_hbm, r1_hbm, offs_hbm,
                   xs_hbm, p0_hbm, p1_hbm,
                   obuf, ibuf, rbuf, p0buf, p1buf, xbuf, sem):
    wid = lax.axis_index("s") * 2 + lax.axis_index("c")
    tok0 = wid * _TPT

    pltpu.sync_copy(offs_hbm, obuf)

    def chunk(c, carry):
        base = tok0 + c * _CH
        for ibh, rbh, pbuf in ((i0_hbm, r0_hbm, p0buf), (i1_hbm, r1_hbm, p1buf)):
            pltpu.sync_copy(ibh.at[pl.ds(base, _CH)], ibuf)
            pltpu.sync_copy(rbh.at[pl.ds(base, _CH)], rbuf)
            for j in range(_CH // 16):
                e16 = ibuf[pl.ds(j * 16, 16)]
                r16 = rbuf[pl.ds(j * 16, 16)]
                o16 = plsc.load_gather(obuf, [e16])
                pbuf[pl.ds(j * 16, 16)] = r16 + o16
        pltpu.sync_copy(p0buf, p0_hbm.at[pl.ds(base, _CH)])
        pltpu.sync_copy(p1buf, p1_hbm.at[pl.ds(base, _CH)])
        pltpu.sync_copy(x_hbm.at[pl.ds(base, _CH)], xbuf)
        pltpu.async_copy(xbuf, xs_hbm.at[p0buf], sem).wait()
        pltpu.async_copy(xbuf, xs_hbm.at[p1buf], sem).wait()
        return carry

    lax.fori_loop(0, _TPT // _CH, chunk, 0)


def _dispatch_call(xf, i0, i1, r0, r1, offs):
    return pl.kernel(
        _dispatch_body,
        out_type=[
            jax.ShapeDtypeStruct((A, D), jnp.float32),
            jax.ShapeDtypeStruct((N_TOK,), jnp.int32),
            jax.ShapeDtypeStruct((N_TOK,), jnp.int32),
        ],
        mesh=_sc_mesh(),
        compiler_params=pltpu.CompilerParams(needs_layout_passes=False),
        scratch_types=[
            pltpu.VMEM((16,), jnp.int32),
            pltpu.VMEM((_CH,), jnp.int32),
            pltpu.VMEM((_CH,), jnp.int32),
            pltpu.VMEM((_CH,), jnp.int32),
            pltpu.VMEM((_CH,), jnp.int32),
            pltpu.VMEM((_CH, D), jnp.float32),
            pltpu.SemaphoreType.DMA,
        ],
    )(xf, i0, i1, r0, r1, offs)


def _ffn_body(t_ref, e_ref, lo_ref, hi_ref,
              xs_ref, w1_ref, b1_ref, w2_ref, b2_ref, out_ref):
    k = pl.program_id(0)
    lo = lo_ref[k]
    hi = hi_ref[k]

    @pl.when(hi > lo)
    def _():
        xb = xs_ref[...]
        h = jax.lax.dot_general(xb, w1_ref[0], (((1,), (1,)), ((), ())),
                                preferred_element_type=jnp.float32)
        h = jnp.maximum(h + b1_ref[0, 0], 0.0).astype(jnp.bfloat16)
        y = jax.lax.dot_general(h, w2_ref[0], (((1,), (1,)), ((), ())),
                                preferred_element_type=jnp.float32)
        t = t_ref[k]
        row = t * TB + lax.broadcasted_iota(jnp.int32, (TB, 1), 0)
        valid = (row >= lo) & (row < hi)
        out_ref[...] = jnp.where(valid, y + b2_ref[0, 0], out_ref[...])


def _ffn_call(wt, we, wlo, whi, xs, W1b, b1, W2b, b2):
    grid_spec = pltpu.PrefetchScalarGridSpec(
        num_scalar_prefetch=4,
        grid=(NW,),
        in_specs=[
            pl.BlockSpec((TB, D), lambda k, t, e, lo, hi: (t[k], 0)),
            pl.BlockSpec((1, DFF, D), lambda k, t, e, lo, hi: (e[k], 0, 0)),
            pl.BlockSpec((1, 1, DFF), lambda k, t, e, lo, hi: (e[k], 0, 0)),
            pl.BlockSpec((1, D, DFF), lambda k, t, e, lo, hi: (e[k], 0, 0)),
            pl.BlockSpec((1, 1, D), lambda k, t, e, lo, hi: (e[k], 0, 0)),
        ],
        out_specs=pl.BlockSpec((TB, D), lambda k, t, e, lo, hi: (t[k], 0)),
    )
    return pl.pallas_call(
        _ffn_body,
        grid_spec=grid_spec,
        out_shape=jax.ShapeDtypeStruct((A, D), jnp.float32),
    )(wt, we, wlo, whi, xs, W1b, b1, W2b, b2)


_CC = 16


def _combine_body(ys_hbm, p0_hbm, p1_hbm, w0_hbm, w1_hbm, out_hbm,
                  pbuf0, pbuf1, abuf, bbuf, obuf, wb0, wb1, sem):
    wid = lax.axis_index("s") * 2 + lax.axis_index("c")
    tok0 = wid * _TPT

    def chunk(c, carry):
        base = tok0 + c * _CC
        pltpu.sync_copy(p0_hbm.at[pl.ds(base, _CC)], pbuf0)
        pltpu.sync_copy(p1_hbm.at[pl.ds(base, _CC)], pbuf1)
        pltpu.sync_copy(w0_hbm.at[pl.ds(base, _CC)], wb0)
        pltpu.sync_copy(w1_hbm.at[pl.ds(base, _CC)], wb1)
        pltpu.async_copy(ys_hbm.at[pbuf0], abuf, sem).wait()
        pltpu.async_copy(ys_hbm.at[pbuf1], bbuf, sem).wait()

        def row(r, carry2):
            ridx = jnp.broadcast_to(r, (16,)).astype(jnp.int32)
            w0v = plsc.load_gather(wb0, [ridx])
            w1v = plsc.load_gather(wb1, [ridx])

            def vec(j, carry3):
                av = abuf[r, pl.ds(j * 16, 16)]
                bv = bbuf[r, pl.ds(j * 16, 16)]
                obuf[r, pl.ds(j * 16, 16)] = av * w0v + bv * w1v
                return carry3

            return lax.fori_loop(0, D // 16, vec, carry2, unroll=4)

        lax.fori_loop(0, _CC, row, 0)
        pltpu.sync_copy(obuf, out_hbm.at[pl.ds(base, _CC)])
        return carry

    lax.fori_loop(0, _TPT // _CC, chunk, 0)


def _combine_call(ys, p0, p1, w0, w1):
    return pl.kernel(
        _combine_body,
        out_type=jax.ShapeDtypeStruct((N_TOK, D), jnp.float32),
        mesh=_sc_mesh(),
        compiler_params=pltpu.CompilerParams(needs_layout_passes=False),
        scratch_types=[
            pltpu.VMEM((_CC,), jnp.int32),
            pltpu.VMEM((_CC,), jnp.int32),
            pltpu.VMEM((_CC, D), jnp.float32),
            pltpu.VMEM((_CC, D), jnp.float32),
            pltpu.VMEM((_CC, D), jnp.float32),
            pltpu.VMEM((_CC,), jnp.float32),
            pltpu.VMEM((_CC,), jnp.float32),
            pltpu.SemaphoreType.DMA,
        ],
    )(ys, p0, p1, w0, w1)


def _schedule(hist):
    off = jnp.concatenate(
        [jnp.zeros((1,), jnp.int32), jnp.cumsum(hist[0, :E], dtype=jnp.int32)])
    c = jnp.arange(NT * E, dtype=jnp.int32)
    t = c // E
    e = c % E
    lo = jnp.maximum(t * TB, off[e])
    hi = jnp.minimum((t + 1) * TB, off[e + 1])
    valid = hi > lo
    slot = jnp.where(valid, jnp.cumsum(valid.astype(jnp.int32)) - 1, NW)
    nvalid = jnp.sum(valid.astype(jnp.int32))
    wt = jnp.zeros((NW + 1,), jnp.int32).at[slot].set(t, mode="drop")
    we = jnp.zeros((NW + 1,), jnp.int32).at[slot].set(e, mode="drop")
    wlo = jnp.zeros((NW + 1,), jnp.int32).at[slot].set(lo, mode="drop")
    whi = jnp.zeros((NW + 1,), jnp.int32).at[slot].set(hi, mode="drop")
    pad = jnp.arange(NW) >= nvalid
    lt = wt[jnp.maximum(nvalid - 1, 0)]
    le = we[jnp.maximum(nvalid - 1, 0)]
    wt = jnp.where(pad, lt, wt[:NW])
    we = jnp.where(pad, le, we[:NW])
    wlo = jnp.where(pad, 0, wlo[:NW])
    whi = jnp.where(pad, 0, whi[:NW])
    offs16 = jnp.concatenate([off[:E], jnp.zeros((16 - E,), jnp.int32)])
    return wt, we, wlo, whi, offs16


def kernel(x, Wg, bg, W1, b1, W2, b2):
    xf = x.reshape(N_TOK, D)
    i0, i1, r0, r1, w0, w1, hist = _gating_call(xf, Wg, bg)
    wt, we, wlo, whi, offs16 = _schedule(hist)
    xs, p0, p1 = _dispatch_call(
        xf, i0.reshape(-1), i1.reshape(-1), r0.reshape(-1), r1.reshape(-1),
        offs16)
    ys = _ffn_call(wt, we, wlo, whi, xs.astype(jnp.bfloat16),
                   W1.astype(jnp.bfloat16), b1.reshape(E, 1, DFF),
                   W2.astype(jnp.bfloat16), b2.reshape(E, 1, D))
    out = _combine_call(ys, p0, p1, w0.reshape(-1), w1.reshape(-1))
    return out.reshape(x.shape)
